# baseline (speedup 1.0000x reference)
import functools

import jax
import jax.numpy as jnp
from jax import lax
from jax.experimental import pallas as pl
from jax.experimental.pallas import tpu as pltpu
from jax.experimental.pallas import tpu_sc as plsc

NC = 2
NS = 16
NW = NC * NS
K = 128


def _sc_agg_body(nch, rpt, h_hbm, src_hbm, dst_hbm, zeros_hbm, agg_hbm,
                 src_v, dst_v, rows_a, acc_sh, sem_a):
    c = lax.axis_index("c")
    s = lax.axis_index("s")
    wid = c * NS + s
    pltpu.sync_copy(zeros_hbm, acc_sh.at[pl.ds(s * rpt, rpt)])
    pltpu.sync_copy(src_hbm.at[wid], src_v)
    pltpu.sync_copy(dst_hbm.at[wid], dst_v)
    plsc.subcore_barrier()

    def chunk(j, _):
        pltpu.async_copy(h_hbm.at[src_v.at[j]], rows_a, sem_a).wait()
        pltpu.sync_copy(rows_a, acc_sh.at[dst_v.at[j]], add=True)
        return _

    lax.fori_loop(0, nch, chunk, 0)
    plsc.subcore_barrier()
    pltpu.sync_copy(acc_sh.at[pl.ds(s * rpt, rpt)], agg_hbm.at[c, pl.ds(s * rpt, rpt)])


@functools.lru_cache(maxsize=None)
def _make_sc_agg(n, d, nch):
    nacc = (n // 128 + 1) * 128
    rpt = nacc // NS
    mesh = plsc.VectorSubcoreMesh(core_axis_name="c", subcore_axis_name="s")
    return pl.kernel(
        functools.partial(_sc_agg_body, nch, rpt),
        out_type=jax.ShapeDtypeStruct((NC, nacc, d), jnp.float32),
        mesh=mesh,
        scratch_types=[
            pltpu.VMEM((nch, K), jnp.int32),
            pltpu.VMEM((nch, K), jnp.int32),
            pltpu.VMEM((K, d), jnp.float32),
            pltpu.VMEM_SHARED((nacc, d), jnp.float32),
            pltpu.SemaphoreType.DMA,
        ],
    )


def _dense_body(h_ref, a0_ref, a1_ref, ws_ref, wa_ref, b_ref, o_ref):
    agg = a0_ref[...] + a1_ref[...]
    acc = jnp.dot(h_ref[...], ws_ref[...], preferred_element_type=jnp.float32)
    acc = acc + jnp.dot(agg, wa_ref[...], preferred_element_type=jnp.float32)
    o_ref[...] = jnp.maximum(acc + b_ref[...], 0.0)


@functools.lru_cache(maxsize=None)
def _make_dense(n, d, bl):
    return pl.pallas_call(
        _dense_body,
        grid=(n // bl,),
        in_specs=[
            pl.BlockSpec((bl, d), lambda i: (i, 0)),
            pl.BlockSpec((bl, d), lambda i: (i, 0)),
            pl.BlockSpec((bl, d), lambda i: (i, 0)),
            pl.BlockSpec((d, d), lambda i: (0, 0)),
            pl.BlockSpec((d, d), lambda i: (0, 0)),
            pl.BlockSpec((1, d), lambda i: (0, 0)),
        ],
        out_specs=pl.BlockSpec((bl, d), lambda i: (i, 0)),
        out_shape=jax.ShapeDtypeStruct((n, d), jnp.float32),
    )


def kernel(e_nodes, edge_index, x_nodes, W_self, W_agg, b):
    B, n, d = e_nodes.shape
    e = edge_index.shape[1]
    num_layers = W_self.shape[0]

    ept = -(-e // NW)
    nch = -(-ept // K)
    nch += nch % 2
    pad = NW * nch * K - e

    src = edge_index[0]
    dst = edge_index[1]
    nacc = (n // 128 + 1) * 128
    pad_dst = n + jnp.arange(pad, dtype=jnp.int32) % (nacc - n)
    src_p = jnp.concatenate([src, jnp.zeros((pad,), jnp.int32)]).reshape(NW, nch, K)
    dst_p = jnp.concatenate([dst, pad_dst]).reshape(NW, nch, K)
    zeros = jnp.zeros((nacc // NS, d), jnp.float32)

    sc_agg = _make_sc_agg(n, d, nch)
    dense = _make_dense(n, d, 1000)

    h = e_nodes[0]
    for l in range(num_layers):
        agg = sc_agg(h, src_p, dst_p, zeros)
        h = dense(h, agg[0], agg[1], W_self[l], W_agg[l], b[l].reshape(1, d))

    out = jnp.concatenate([x_nodes[..., :4], h[None]], axis=2)
    return (out, edge_index)


# speedup vs baseline: 1.4939x; 1.4939x over previous
import functools

import jax
import jax.numpy as jnp
from jax import lax
from jax.experimental import pallas as pl
from jax.experimental.pallas import tpu as pltpu
from jax.experimental.pallas import tpu_sc as plsc

NC = 2
NS = 16
NW = NC * NS
K = 128


def _sc_agg_body(nch, rpt, h_hbm, src_hbm, dst_hbm, zeros_hbm, agg_hbm,
                 src_v, dst_v, rows_a, acc_sh, sem_a):
    c = lax.axis_index("c")
    s = lax.axis_index("s")
    wid = c * NS + s
    pltpu.sync_copy(zeros_hbm, acc_sh.at[pl.ds(s * rpt, rpt)])
    pltpu.sync_copy(src_hbm.at[wid], src_v)
    pltpu.sync_copy(dst_hbm.at[wid], dst_v)
    plsc.subcore_barrier()

    def chunk(j, _):
        pltpu.async_copy(h_hbm.at[src_v.at[j]], rows_a, sem_a).wait()
        pltpu.sync_copy(rows_a, acc_sh.at[dst_v.at[j]], add=True)
        return _

    lax.fori_loop(0, nch, chunk, 0)
    plsc.subcore_barrier()
    pltpu.sync_copy(acc_sh.at[pl.ds(s * rpt, rpt)], agg_hbm.at[c, pl.ds(s * rpt, rpt)])


@functools.lru_cache(maxsize=None)
def _make_sc_agg(n, d, nch):
    nacc = (n // 128 + 1) * 128
    rpt = nacc // NS
    mesh = plsc.VectorSubcoreMesh(core_axis_name="c", subcore_axis_name="s")
    return pl.kernel(
        functools.partial(_sc_agg_body, nch, rpt),
        out_type=jax.ShapeDtypeStruct((NC, nacc, d), jnp.float32),
        mesh=mesh,
        scratch_types=[
            pltpu.VMEM((nch, K), jnp.int32),
            pltpu.VMEM((nch, K), jnp.int32),
            pltpu.VMEM((K, d), jnp.float32),
            pltpu.VMEM_SHARED((nacc, d), jnp.float32),
            pltpu.SemaphoreType.DMA,
        ],
    )


def _dense_body(h_ref, a0_ref, a1_ref, ws_ref, wa_ref, b_ref, o_ref):
    agg = a0_ref[...] + a1_ref[...]
    acc = jnp.dot(h_ref[...], ws_ref[...], preferred_element_type=jnp.float32)
    acc = acc + jnp.dot(agg, wa_ref[...], preferred_element_type=jnp.float32)
    o_ref[...] = jnp.maximum(acc + b_ref[...], 0.0)


@functools.lru_cache(maxsize=None)
def _make_dense(n, d, bl):
    return pl.pallas_call(
        _dense_body,
        grid=(n // bl,),
        in_specs=[
            pl.BlockSpec((bl, d), lambda i: (i, 0)),
            pl.BlockSpec((bl, d), lambda i: (i, 0)),
            pl.BlockSpec((bl, d), lambda i: (i, 0)),
            pl.BlockSpec((d, d), lambda i: (0, 0)),
            pl.BlockSpec((d, d), lambda i: (0, 0)),
            pl.BlockSpec((1, d), lambda i: (0, 0)),
        ],
        out_specs=pl.BlockSpec((bl, d), lambda i: (i, 0)),
        out_shape=jax.ShapeDtypeStruct((n, d), jnp.float32),
    )


def kernel(e_nodes, edge_index, x_nodes, W_self, W_agg, b):
    B, n, d = e_nodes.shape
    e = edge_index.shape[1]
    num_layers = W_self.shape[0]

    ept = -(-e // NW)
    nch = -(-ept // K)
    pad = NW * nch * K - e

    src = edge_index[0]
    dst = edge_index[1]
    nacc = (n // 128 + 1) * 128
    pad_dst = n + jnp.arange(pad, dtype=jnp.int32) % (nacc - n)
    src_p = jnp.concatenate([src, jnp.zeros((pad,), jnp.int32)]).reshape(NW, nch, K)
    dst_p = jnp.concatenate([dst, pad_dst]).reshape(NW, nch, K)
    zeros = jnp.zeros((nacc // NS, d), jnp.float32)

    sc_agg = _make_sc_agg(n, d, nch)
    dense = _make_dense(n, d, 1000)

    h = e_nodes[0]
    for l in range(num_layers):
        agg = sc_agg(h, src_p, dst_p, zeros)
        h = dense(h, agg[0], agg[1], W_self[l], W_agg[l], b[l].reshape(1, d))

    out = jnp.concatenate([x_nodes[..., :4], h[None]], axis=2)
    return (out, edge_index)


# speedup vs baseline: 2.5022x; 1.6749x over previous
import functools

import jax
import jax.numpy as jnp
from jax import lax
from jax.experimental import pallas as pl
from jax.experimental.pallas import tpu as pltpu
from jax.experimental.pallas import tpu_sc as plsc

NC = 2
NS = 16
NW = NC * NS
K = 128


def _sc_agg_body(nch, rpt, h_hbm, src_hbm, dst_hbm, zeros_hbm, agg_hbm,
                 src_v, dst_v, rows_a, acc_sh, sem_a):
    c = lax.axis_index("c")
    s = lax.axis_index("s")
    wid = c * NS + s
    pltpu.sync_copy(zeros_hbm, acc_sh.at[pl.ds(s * rpt, rpt)])
    pltpu.sync_copy(src_hbm.at[wid], src_v)
    pltpu.sync_copy(dst_hbm.at[wid], dst_v)
    plsc.subcore_barrier()

    def chunk(j, _):
        pltpu.async_copy(h_hbm.at[src_v.at[j]], rows_a, sem_a).wait()
        pltpu.sync_copy(rows_a, acc_sh.at[dst_v.at[j]], add=True)
        return _

    lax.fori_loop(0, nch, chunk, 0)
    plsc.subcore_barrier()
    pltpu.sync_copy(acc_sh.at[pl.ds(s * rpt, rpt)], agg_hbm.at[c, pl.ds(s * rpt, rpt)])


@functools.lru_cache(maxsize=None)
def _make_sc_agg(n, d, nch):
    nacc = (n // 128 + 1) * 128
    rpt = nacc // NS
    mesh = plsc.VectorSubcoreMesh(core_axis_name="c", subcore_axis_name="s")
    return pl.kernel(
        functools.partial(_sc_agg_body, nch, rpt),
        out_type=jax.ShapeDtypeStruct((NC, nacc, d), jnp.float32),
        mesh=mesh,
        scratch_types=[
            pltpu.VMEM((nch, K), jnp.int32),
            pltpu.VMEM((nch, K), jnp.int32),
            pltpu.VMEM((K, d), jnp.float32),
            pltpu.VMEM_SHARED((nacc, d), jnp.float32),
            pltpu.SemaphoreType.DMA,
        ],
    )


def _dense_body(h_ref, a0_ref, a1_ref, ws_ref, wa_ref, b_ref, o_ref):
    agg = a0_ref[...] + a1_ref[...]
    acc = jnp.dot(h_ref[...], ws_ref[...], preferred_element_type=jnp.float32)
    acc = acc + jnp.dot(agg, wa_ref[...], preferred_element_type=jnp.float32)
    o_ref[...] = jnp.maximum(acc + b_ref[...], 0.0)


@functools.lru_cache(maxsize=None)
def _make_dense(n, d, bl):
    return pl.pallas_call(
        _dense_body,
        grid=(n // bl,),
        in_specs=[
            pl.BlockSpec((bl, d), lambda i: (i, 0)),
            pl.BlockSpec((bl, d), lambda i: (i, 0)),
            pl.BlockSpec((bl, d), lambda i: (i, 0)),
            pl.BlockSpec((d, d), lambda i: (0, 0)),
            pl.BlockSpec((d, d), lambda i: (0, 0)),
            pl.BlockSpec((1, d), lambda i: (0, 0)),
        ],
        out_specs=pl.BlockSpec((bl, d), lambda i: (i, 0)),
        out_shape=jax.ShapeDtypeStruct((n, d), jnp.float32),
    )


def kernel(e_nodes, edge_index, x_nodes, W_self, W_agg, b):
    B, n, d = e_nodes.shape
    e = edge_index.shape[1]
    num_layers = W_self.shape[0]

    ept = -(-e // NW)
    nch = -(-ept // K)
    nch += nch % 2
    pad = NW * nch * K - e

    src = edge_index[0]
    dst = edge_index[1]
    nacc = (n // 128 + 1) * 128
    pad_dst = n + jnp.arange(pad, dtype=jnp.int32) % (nacc - n)
    pad_src = jnp.arange(pad, dtype=jnp.int32) % n
    src_p = jnp.concatenate([src, pad_src]).reshape(NW, nch, K)
    dst_p = jnp.concatenate([dst, pad_dst]).reshape(NW, nch, K)
    zeros = jnp.zeros((nacc // NS, d), jnp.float32)

    sc_agg = _make_sc_agg(n, d, nch)
    dense = _make_dense(n, d, 1000)

    h = e_nodes[0]
    for l in range(num_layers):
        agg = sc_agg(h, src_p, dst_p, zeros)
        h = dense(h, agg[0], agg[1], W_self[l], W_agg[l], b[l].reshape(1, d))

    out = jnp.concatenate([x_nodes[..., :4], h[None]], axis=2)
    return (out, edge_index)


# speedup vs baseline: 3.1802x; 1.2710x over previous
import functools

import jax
import jax.numpy as jnp
from jax import lax
from jax.experimental import pallas as pl
from jax.experimental.pallas import tpu as pltpu
from jax.experimental.pallas import tpu_sc as plsc

NC = 2
NS = 16
NW = NC * NS
K = 128


def _sc_agg_body(nch, rpt, h_hbm, src_hbm, dst_hbm, zeros_hbm, agg_hbm,
                 src_v, dst_v, rows_a, rows_b, acc_sh, sem_a, sem_b):
    c = lax.axis_index("c")
    s = lax.axis_index("s")
    wid = c * NS + s
    nwin = nch // 2
    pltpu.sync_copy(zeros_hbm, acc_sh.at[pl.ds(s * rpt, rpt)])
    plsc.subcore_barrier()

    def do_window(w):
        pltpu.sync_copy(src_hbm.at[wid, pl.ds(w * nwin, nwin)], src_v)
        pltpu.sync_copy(dst_hbm.at[wid, pl.ds(w * nwin, nwin)], dst_v)
        pltpu.async_copy(h_hbm.at[src_v.at[0]], rows_a, sem_a)

        def chunk2(i, _):
            j = 2 * i
            pltpu.make_async_copy(h_hbm.at[src_v.at[j]], rows_a, sem_a).wait()
            pltpu.async_copy(h_hbm.at[src_v.at[j + 1]], rows_b, sem_b)
            pltpu.sync_copy(rows_a, acc_sh.at[dst_v.at[j]], add=True)
            pltpu.make_async_copy(h_hbm.at[src_v.at[j + 1]], rows_b, sem_b).wait()

            @pl.when(j + 2 < nwin)
            def _start_next():
                pltpu.async_copy(h_hbm.at[src_v.at[j + 2]], rows_a, sem_a)

            pltpu.sync_copy(rows_b, acc_sh.at[dst_v.at[j + 1]], add=True)
            return _

        lax.fori_loop(0, nwin // 2, chunk2, 0)

    do_window(0)
    do_window(1)
    plsc.subcore_barrier()
    pltpu.sync_copy(acc_sh.at[pl.ds(s * rpt, rpt)], agg_hbm.at[c, pl.ds(s * rpt, rpt)])


@functools.lru_cache(maxsize=None)
def _make_sc_agg(n, d, nch):
    nacc = (n // 128 + 1) * 128
    rpt = nacc // NS
    mesh = plsc.VectorSubcoreMesh(core_axis_name="c", subcore_axis_name="s")
    return pl.kernel(
        functools.partial(_sc_agg_body, nch, rpt),
        out_type=jax.ShapeDtypeStruct((NC, nacc, d), jnp.float32),
        mesh=mesh,
        scratch_types=[
            pltpu.VMEM((nch // 2, K), jnp.int32),
            pltpu.VMEM((nch // 2, K), jnp.int32),
            pltpu.VMEM((K, d), jnp.float32),
            pltpu.VMEM((K, d), jnp.float32),
            pltpu.VMEM_SHARED((nacc, d), jnp.float32),
            pltpu.SemaphoreType.DMA,
            pltpu.SemaphoreType.DMA,
        ],
    )


def _dense_body(h_ref, a0_ref, a1_ref, ws_ref, wa_ref, b_ref, o_ref):
    agg = a0_ref[...] + a1_ref[...]
    acc = jnp.dot(h_ref[...], ws_ref[...], preferred_element_type=jnp.float32)
    acc = acc + jnp.dot(agg, wa_ref[...], preferred_element_type=jnp.float32)
    o_ref[...] = jnp.maximum(acc + b_ref[...], 0.0)


@functools.lru_cache(maxsize=None)
def _make_dense(n, d, bl):
    return pl.pallas_call(
        _dense_body,
        grid=(n // bl,),
        in_specs=[
            pl.BlockSpec((bl, d), lambda i: (i, 0)),
            pl.BlockSpec((bl, d), lambda i: (i, 0)),
            pl.BlockSpec((bl, d), lambda i: (i, 0)),
            pl.BlockSpec((d, d), lambda i: (0, 0)),
            pl.BlockSpec((d, d), lambda i: (0, 0)),
            pl.BlockSpec((1, d), lambda i: (0, 0)),
        ],
        out_specs=pl.BlockSpec((bl, d), lambda i: (i, 0)),
        out_shape=jax.ShapeDtypeStruct((n, d), jnp.float32),
    )


def kernel(e_nodes, edge_index, x_nodes, W_self, W_agg, b):
    B, n, d = e_nodes.shape
    e = edge_index.shape[1]
    num_layers = W_self.shape[0]

    ept = -(-e // NW)
    nch = -(-ept // K)
    nch += nch % 2
    pad = NW * nch * K - e

    src = edge_index[0]
    dst = edge_index[1]
    nacc = (n // 128 + 1) * 128
    pad_dst = n + jnp.arange(pad, dtype=jnp.int32) % (nacc - n)
    pad_src = jnp.arange(pad, dtype=jnp.int32) % n
    src_p = jnp.concatenate([src, pad_src]).reshape(NW, nch, K)
    dst_p = jnp.concatenate([dst, pad_dst]).reshape(NW, nch, K)
    zeros = jnp.zeros((nacc // NS, d), jnp.float32)

    sc_agg = _make_sc_agg(n, d, nch)
    dense = _make_dense(n, d, 1000)

    h = e_nodes[0]
    for l in range(num_layers):
        agg = sc_agg(h, src_p, dst_p, zeros)
        h = dense(h, agg[0], agg[1], W_self[l], W_agg[l], b[l].reshape(1, d))

    out = jnp.concatenate([x_nodes[..., :4], h[None]], axis=2)
    return (out, edge_index)


# speedup vs baseline: 3.3342x; 1.0484x over previous
import functools

import jax
import jax.numpy as jnp
from jax import lax
from jax.experimental import pallas as pl
from jax.experimental.pallas import tpu as pltpu
from jax.experimental.pallas import tpu_sc as plsc

NC = 2
NS = 16
NW = NC * NS
K = 128


def _sc_agg_body(nch, rpt, h_hbm, src_hbm, dst_hbm, zeros_hbm, agg_hbm,
                 src_v, dst_v, rows_a, rows_b, acc_sh, sem_a, sem_b):
    c = lax.axis_index("c")
    s = lax.axis_index("s")
    wid = c * NS + s
    nwin = nch // 2
    pltpu.sync_copy(zeros_hbm, acc_sh.at[pl.ds(s * rpt, rpt)])
    plsc.subcore_barrier()

    def do_window(w):
        pltpu.sync_copy(src_hbm.at[wid, pl.ds(w * nwin, nwin)], src_v)
        pltpu.sync_copy(dst_hbm.at[wid, pl.ds(w * nwin, nwin)], dst_v)
        pltpu.async_copy(h_hbm.at[src_v.at[0]], rows_a, sem_a)

        def chunk2(i, _):
            j = 2 * i
            pltpu.make_async_copy(h_hbm.at[src_v.at[j]], rows_a, sem_a).wait()
            pltpu.async_copy(h_hbm.at[src_v.at[j + 1]], rows_b, sem_b)
            pltpu.sync_copy(rows_a, acc_sh.at[dst_v.at[j]], add=True)
            pltpu.make_async_copy(h_hbm.at[src_v.at[j + 1]], rows_b, sem_b).wait()

            @pl.when(j + 2 < nwin)
            def _start_next():
                pltpu.async_copy(h_hbm.at[src_v.at[j + 2]], rows_a, sem_a)

            pltpu.sync_copy(rows_b, acc_sh.at[dst_v.at[j + 1]], add=True)
            return _

        lax.fori_loop(0, nwin // 2, chunk2, 0)

    do_window(0)
    do_window(1)
    plsc.subcore_barrier()
    pltpu.sync_copy(acc_sh.at[pl.ds(s * rpt, rpt)], agg_hbm.at[c, pl.ds(s * rpt, rpt)])


@functools.lru_cache(maxsize=None)
def _make_sc_agg(n, d, nch):
    nacc = (n // 128 + 1) * 128
    rpt = nacc // NS
    mesh = plsc.VectorSubcoreMesh(core_axis_name="c", subcore_axis_name="s")
    return pl.kernel(
        functools.partial(_sc_agg_body, nch, rpt),
        out_type=jax.ShapeDtypeStruct((NC, nacc, d), jnp.float32),
        mesh=mesh,
        scratch_types=[
            pltpu.VMEM((nch // 2, K), jnp.int32),
            pltpu.VMEM((nch // 2, K), jnp.int32),
            pltpu.VMEM((K, d), jnp.float32),
            pltpu.VMEM((K, d), jnp.float32),
            pltpu.VMEM_SHARED((nacc, d), jnp.float32),
            pltpu.SemaphoreType.DMA,
            pltpu.SemaphoreType.DMA,
        ],
    )


def _dense_body(h_ref, a0_ref, a1_ref, ws_ref, wa_ref, b_ref, o_ref):
    agg = a0_ref[0] + a1_ref[0]
    acc = jnp.dot(h_ref[...], ws_ref[...], preferred_element_type=jnp.float32)
    acc = acc + jnp.dot(agg, wa_ref[...], preferred_element_type=jnp.float32)
    o_ref[...] = jnp.maximum(acc + b_ref[...], 0.0)


@functools.lru_cache(maxsize=None)
def _make_dense(n, d, bl):
    return pl.pallas_call(
        _dense_body,
        grid=(n // bl,),
        in_specs=[
            pl.BlockSpec((bl, d), lambda i: (i, 0)),
            pl.BlockSpec((1, bl, d), lambda i: (0, i, 0)),
            pl.BlockSpec((1, bl, d), lambda i: (1, i, 0)),
            pl.BlockSpec((d, d), lambda i: (0, 0)),
            pl.BlockSpec((d, d), lambda i: (0, 0)),
            pl.BlockSpec((1, d), lambda i: (0, 0)),
        ],
        out_specs=pl.BlockSpec((bl, d), lambda i: (i, 0)),
        out_shape=jax.ShapeDtypeStruct((n, d), jnp.float32),
    )


def kernel(e_nodes, edge_index, x_nodes, W_self, W_agg, b):
    B, n, d = e_nodes.shape
    e = edge_index.shape[1]
    num_layers = W_self.shape[0]

    ept = -(-e // NW)
    nch = -(-ept // K)
    nch += nch % 2
    pad = NW * nch * K - e

    src = edge_index[0]
    dst = edge_index[1]
    nacc = (n // 128 + 1) * 128
    pad_dst = n + jnp.arange(pad, dtype=jnp.int32) % (nacc - n)
    pad_src = jnp.arange(pad, dtype=jnp.int32) % n
    src_p = jnp.concatenate([src, pad_src]).reshape(NW, nch, K)
    dst_p = jnp.concatenate([dst, pad_dst]).reshape(NW, nch, K)
    zeros = jnp.zeros((nacc // NS, d), jnp.float32)

    sc_agg = _make_sc_agg(n, d, nch)
    dense = _make_dense(n, d, 1000)

    h = e_nodes[0]
    for l in range(num_layers):
        agg = sc_agg(h, src_p, dst_p, zeros)
        h = dense(h, agg, agg, W_self[l], W_agg[l], b[l].reshape(1, d))

    out = jnp.concatenate([x_nodes[..., :4], h[None]], axis=2)
    return (out, edge_index)


# speedup vs baseline: 3.7520x; 1.1253x over previous
import functools

import jax
import jax.numpy as jnp
from jax import lax
from jax.experimental import pallas as pl
from jax.experimental.pallas import tpu as pltpu
from jax.experimental.pallas import tpu_sc as plsc

NC = 2
NS = 16
NW = NC * NS
K = 128


def _sc_agg_body(nch, rpt, h_hbm, src_hbm, dst_hbm, zeros_hbm, agg_hbm,
                 src_v, dst_v, rows_a, rows_b, acc_sh, sem_a, sem_b):
    c = lax.axis_index("c")
    s = lax.axis_index("s")
    wid = c * NS + s
    nwin = nch // 2
    pltpu.sync_copy(zeros_hbm, acc_sh.at[pl.ds(s * rpt, rpt)])
    plsc.subcore_barrier()

    def do_window(w):
        pltpu.sync_copy(src_hbm.at[wid, pl.ds(w * nwin, nwin)], src_v)
        pltpu.sync_copy(dst_hbm.at[wid, pl.ds(w * nwin, nwin)], dst_v)
        pltpu.async_copy(h_hbm.at[src_v.at[0]], rows_a, sem_a)
        pltpu.async_copy(h_hbm.at[src_v.at[1]], rows_b, sem_b)

        def chunk2(i, _):
            j = 2 * i
            pltpu.make_async_copy(h_hbm.at[src_v.at[j]], rows_a, sem_a).wait()
            pltpu.sync_copy(rows_a, acc_sh.at[dst_v.at[j]], add=True)

            @pl.when(j + 2 < nwin)
            def _next_a():
                pltpu.async_copy(h_hbm.at[src_v.at[j + 2]], rows_a, sem_a)

            pltpu.make_async_copy(h_hbm.at[src_v.at[j + 1]], rows_b, sem_b).wait()
            pltpu.sync_copy(rows_b, acc_sh.at[dst_v.at[j + 1]], add=True)

            @pl.when(j + 3 < nwin)
            def _next_b():
                pltpu.async_copy(h_hbm.at[src_v.at[j + 3]], rows_b, sem_b)

            return _

        lax.fori_loop(0, nwin // 2, chunk2, 0)

    do_window(0)
    do_window(1)
    plsc.subcore_barrier()
    pltpu.sync_copy(acc_sh.at[pl.ds(s * rpt, rpt)], agg_hbm.at[c, pl.ds(s * rpt, rpt)])


@functools.lru_cache(maxsize=None)
def _make_sc_agg(n, d, nch):
    nacc = (n // 128 + 1) * 128
    rpt = nacc // NS
    mesh = plsc.VectorSubcoreMesh(core_axis_name="c", subcore_axis_name="s")
    return pl.kernel(
        functools.partial(_sc_agg_body, nch, rpt),
        out_type=jax.ShapeDtypeStruct((NC, nacc, d), jnp.float32),
        mesh=mesh,
        scratch_types=[
            pltpu.VMEM((nch // 2, K), jnp.int32),
            pltpu.VMEM((nch // 2, K), jnp.int32),
            pltpu.VMEM((K, d), jnp.float32),
            pltpu.VMEM((K, d), jnp.float32),
            pltpu.VMEM_SHARED((nacc, d), jnp.float32),
            pltpu.SemaphoreType.DMA,
            pltpu.SemaphoreType.DMA,
        ],
    )


def _dense_body(h_ref, a0_ref, a1_ref, ws_ref, wa_ref, b_ref, o_ref):
    agg = a0_ref[0] + a1_ref[0]
    acc = jnp.dot(h_ref[...], ws_ref[...], preferred_element_type=jnp.float32)
    acc = acc + jnp.dot(agg, wa_ref[...], preferred_element_type=jnp.float32)
    o_ref[...] = jnp.maximum(acc + b_ref[...], 0.0)


@functools.lru_cache(maxsize=None)
def _make_dense(n, d, bl):
    return pl.pallas_call(
        _dense_body,
        grid=(n // bl,),
        in_specs=[
            pl.BlockSpec((bl, d), lambda i: (i, 0)),
            pl.BlockSpec((1, bl, d), lambda i: (0, i, 0)),
            pl.BlockSpec((1, bl, d), lambda i: (1, i, 0)),
            pl.BlockSpec((d, d), lambda i: (0, 0)),
            pl.BlockSpec((d, d), lambda i: (0, 0)),
            pl.BlockSpec((1, d), lambda i: (0, 0)),
        ],
        out_specs=pl.BlockSpec((bl, d), lambda i: (i, 0)),
        out_shape=jax.ShapeDtypeStruct((n, d), jnp.float32),
    )


def kernel(e_nodes, edge_index, x_nodes, W_self, W_agg, b):
    B, n, d = e_nodes.shape
    e = edge_index.shape[1]
    num_layers = W_self.shape[0]

    ept = -(-e // NW)
    nch = -(-ept // K)
    nch += nch % 2
    pad = NW * nch * K - e

    src = edge_index[0]
    dst = edge_index[1]
    nacc = (n // 128 + 1) * 128
    pad_dst = n + jnp.arange(pad, dtype=jnp.int32) % (nacc - n)
    pad_src = jnp.arange(pad, dtype=jnp.int32) % n
    src_p = jnp.concatenate([src, pad_src]).reshape(NW, nch, K)
    dst_p = jnp.concatenate([dst, pad_dst]).reshape(NW, nch, K)
    zeros = jnp.zeros((nacc // NS, d), jnp.float32)

    sc_agg = _make_sc_agg(n, d, nch)
    dense = _make_dense(n, d, 1000)

    h = e_nodes[0]
    for l in range(num_layers):
        agg = sc_agg(h, src_p, dst_p, zeros)
        h = dense(h, agg, agg, W_self[l], W_agg[l], b[l].reshape(1, d))

    out = jnp.concatenate([x_nodes[..., :4], h[None]], axis=2)
    return (out, edge_index)


# speedup vs baseline: 3.8718x; 1.0319x over previous
import functools

import jax
import jax.numpy as jnp
from jax import lax
from jax.experimental import pallas as pl
from jax.experimental.pallas import tpu as pltpu
from jax.experimental.pallas import tpu_sc as plsc

NC = 2
NS = 16
NW = NC * NS
K = 128


def _sc_agg_body(nch, rpt, h_hbm, src_hbm, dst_hbm, zeros_hbm, agg_hbm,
                 src_v, dst_v, rows_a, rows_b, acc_sh, sem_a, sem_b):
    c = lax.axis_index("c")
    s = lax.axis_index("s")
    wid = c * NS + s
    nwin = nch // 2
    pltpu.sync_copy(zeros_hbm, acc_sh.at[pl.ds(s * rpt, rpt)])
    plsc.subcore_barrier()

    def do_window(w):
        pltpu.sync_copy(src_hbm.at[wid, pl.ds(w * nwin, nwin)], src_v)
        pltpu.async_copy(h_hbm.at[src_v.at[0]], rows_a, sem_a)
        pltpu.async_copy(h_hbm.at[src_v.at[1]], rows_b, sem_b)
        pltpu.sync_copy(dst_hbm.at[wid, pl.ds(w * nwin, nwin)], dst_v)

        def chunk2(i, _):
            j = 2 * i
            pltpu.make_async_copy(h_hbm.at[src_v.at[j]], rows_a, sem_a).wait()
            pltpu.sync_copy(rows_a, acc_sh.at[dst_v.at[j]], add=True)

            @pl.when(j + 2 < nwin)
            def _next_a():
                pltpu.async_copy(h_hbm.at[src_v.at[j + 2]], rows_a, sem_a)

            pltpu.make_async_copy(h_hbm.at[src_v.at[j + 1]], rows_b, sem_b).wait()
            pltpu.sync_copy(rows_b, acc_sh.at[dst_v.at[j + 1]], add=True)

            @pl.when(j + 3 < nwin)
            def _next_b():
                pltpu.async_copy(h_hbm.at[src_v.at[j + 3]], rows_b, sem_b)

            return _

        lax.fori_loop(0, nwin // 2, chunk2, 0)

    do_window(0)
    do_window(1)
    plsc.subcore_barrier()
    pltpu.sync_copy(acc_sh.at[pl.ds(s * rpt, rpt)], agg_hbm.at[c, pl.ds(s * rpt, rpt)])


@functools.lru_cache(maxsize=None)
def _make_sc_agg(n, d, nch):
    nacc = (n // 128 + 1) * 128
    rpt = nacc // NS
    mesh = plsc.VectorSubcoreMesh(core_axis_name="c", subcore_axis_name="s")
    return pl.kernel(
        functools.partial(_sc_agg_body, nch, rpt),
        out_type=jax.ShapeDtypeStruct((NC, nacc, d), jnp.float32),
        mesh=mesh,
        scratch_types=[
            pltpu.VMEM((nch // 2, K), jnp.int32),
            pltpu.VMEM((nch // 2, K), jnp.int32),
            pltpu.VMEM((K, d), jnp.float32),
            pltpu.VMEM((K, d), jnp.float32),
            pltpu.VMEM_SHARED((nacc, d), jnp.float32),
            pltpu.SemaphoreType.DMA,
            pltpu.SemaphoreType.DMA,
        ],
    )


def _dense_body(h_ref, a0_ref, a1_ref, ws_ref, wa_ref, b_ref, o_ref):
    agg = a0_ref[0] + a1_ref[0]
    acc = jnp.dot(h_ref[...], ws_ref[...], preferred_element_type=jnp.float32)
    acc = acc + jnp.dot(agg, wa_ref[...], preferred_element_type=jnp.float32)
    o_ref[...] = jnp.maximum(acc + b_ref[...], 0.0)


@functools.lru_cache(maxsize=None)
def _make_dense(n, d, bl):
    return pl.pallas_call(
        _dense_body,
        grid=(n // bl,),
        in_specs=[
            pl.BlockSpec((bl, d), lambda i: (i, 0)),
            pl.BlockSpec((1, bl, d), lambda i: (0, i, 0)),
            pl.BlockSpec((1, bl, d), lambda i: (1, i, 0)),
            pl.BlockSpec((d, d), lambda i: (0, 0)),
            pl.BlockSpec((d, d), lambda i: (0, 0)),
            pl.BlockSpec((1, d), lambda i: (0, 0)),
        ],
        out_specs=pl.BlockSpec((bl, d), lambda i: (i, 0)),
        out_shape=jax.ShapeDtypeStruct((n, d), jnp.float32),
    )


def kernel(e_nodes, edge_index, x_nodes, W_self, W_agg, b):
    B, n, d = e_nodes.shape
    e = edge_index.shape[1]
    num_layers = W_self.shape[0]

    ept = -(-e // NW)
    nch = -(-ept // K)
    nch += nch % 2
    pad = NW * nch * K - e

    src = edge_index[0]
    dst = edge_index[1]
    nacc = (n // 128 + 1) * 128
    pad_dst = n + jnp.arange(pad, dtype=jnp.int32) % (nacc - n)
    pad_src = jnp.arange(pad, dtype=jnp.int32) % n
    src_p = jnp.concatenate([src, pad_src]).reshape(NW, nch, K)
    dst_p = jnp.concatenate([dst, pad_dst]).reshape(NW, nch, K)
    zeros = jnp.zeros((nacc // NS, d), jnp.float32)

    sc_agg = _make_sc_agg(n, d, nch)
    dense = _make_dense(n, d, 1000)

    h = e_nodes[0]
    for l in range(num_layers):
        agg = sc_agg(h, src_p, dst_p, zeros)
        h = dense(h, agg, agg, W_self[l], W_agg[l], b[l].reshape(1, d))

    out = jnp.concatenate([x_nodes[..., :4], h[None]], axis=2)
    return (out, edge_index)
